# baseline (speedup 1.0000x reference)
import functools

import jax
import jax.numpy as jnp
from jax import lax
from jax.experimental import pallas as pl
from jax.experimental.pallas import tpu as pltpu
from jax.experimental.pallas import tpu_sc as plsc

_N = 10000
_E = 320000
_D = 128
_NC = 2
_NS = 16
_NW = _NC * _NS
_EPW = _E // _NW
_CH = 40
_NIT = _EPW // _CH
_NPH = 2
_NITH = _NIT // _NPH
_CHS = 64
_EPWP = 10240
_EPAD = _EPWP - _EPW
_NPHS = 2
_NITSH = _EPWP // _CHS // _NPHS
_NP = 10240
_RPW = _NP // _NS
_DW = 128
_DO = 16

@functools.lru_cache(maxsize=None)
def _sc_mesh():
    return plsc.VectorSubcoreMesh(core_axis_name="c", subcore_axis_name="s",
                                  num_cores=_NC, num_subcores=_NS)


def _deg_body(col4_hbm, ones_hbm, zeros_hbm, degp_hbm,
              cidx_half, ones_v, isem, ssem, deg_sp):
    c = lax.axis_index("c")
    s = lax.axis_index("s")
    wid = s * _NC + c
    pltpu.sync_copy(zeros_hbm.at[pl.ds(s * _RPW, _RPW)],
                    deg_sp.at[pl.ds(s * _RPW, _RPW)])
    pltpu.sync_copy(ones_hbm, ones_v)
    pltpu.sync_copy(col4_hbm.at[wid, 0], cidx_half)
    plsc.subcore_barrier()
    for ph in range(_NPH):
        pltpu.async_copy(ones_v, deg_sp.at[cidx_half.at[0]], ssem, add=True)
        pltpu.async_copy(ones_v, deg_sp.at[cidx_half.at[1]], ssem, add=True)

        def step(i, carry):
            pltpu.make_async_copy(ones_v, deg_sp.at[cidx_half.at[i]], ssem).wait()
            pltpu.async_copy(ones_v, deg_sp.at[cidx_half.at[i]], ssem, add=True)
            return carry

        lax.fori_loop(2, _NITH, step, 0)
        pltpu.make_async_copy(ones_v, deg_sp.at[cidx_half.at[0]], ssem).wait()
        pltpu.make_async_copy(ones_v, deg_sp.at[cidx_half.at[0]], ssem).wait()
        if ph + 1 < _NPH:
            pltpu.sync_copy(col4_hbm.at[wid, ph + 1], cidx_half)
    plsc.subcore_barrier()
    pltpu.sync_copy(deg_sp.at[pl.ds(s * _RPW, _RPW)],
                    degp_hbm.at[c, pl.ds(s * _RPW, _RPW)])


@functools.lru_cache(maxsize=None)
def _make_deg_kernel(interpret=False):
    return pl.kernel(
        _deg_body,
        out_type=jax.ShapeDtypeStruct((_NC, _NP, _DW), jnp.float32),
        mesh=_sc_mesh(),
        scratch_types=[
            pltpu.VMEM((_NITH, _CH), jnp.int32),
            pltpu.VMEM((_CH, _DW), jnp.float32),
            pltpu.SemaphoreType.DMA,
            pltpu.SemaphoreType.DMA,
            pltpu.VMEM_SHARED((_NP, _DW), jnp.float32),
        ],
        interpret=interpret,
    )


def _spmm_body(row4_hbm, col_hbm, y_hbm, zeros_hbm, accp_hbm,
               cidx_all, ridx_half, rows0, rows1, rows2,
               isem0, isem1, gs0, gs1, gs2, ss0, ss1, ss2, acc_sp):
    c = lax.axis_index("c")
    s = lax.axis_index("s")
    wid = s * _NC + c
    pltpu.async_copy(col_hbm.at[pl.ds(wid * _EPWP, _EPWP)], cidx_all, isem0)
    pltpu.async_copy(row4_hbm.at[wid, 0], ridx_half, isem1)
    pltpu.sync_copy(zeros_hbm.at[pl.ds(s * _RPW, _RPW)],
                    acc_sp.at[pl.ds(s * _RPW, _RPW)])
    pltpu.make_async_copy(col_hbm.at[pl.ds(wid * _EPWP, _EPWP)], cidx_all,
                          isem0).wait()
    pltpu.make_async_copy(row4_hbm.at[wid, 0], ridx_half, isem1).wait()
    plsc.subcore_barrier()

    rows = (rows0, rows1, rows2)
    gsem = (gs0, gs1, gs2)
    ssem = (ss0, ss1, ss2)

    def gidx(k):
        return cidx_all.at[pl.ds(k * _CHS, _CHS)]

    def fire_g(q, b, cb):
        pltpu.async_copy(y_hbm.at[gidx(cb + q)], rows[b], gsem[b])

    def wait_g(q, b, cb):
        pltpu.make_async_copy(y_hbm.at[gidx(cb + q)], rows[b], gsem[b]).wait()

    def fire_s(i, b):
        pltpu.async_copy(rows[b], acc_sp.at[ridx_half.at[i]], ssem[b],
                         add=True)

    def wait_s(i, b):
        pltpu.make_async_copy(rows[b], acc_sp.at[ridx_half.at[i]],
                              ssem[b]).wait()

    for ph in range(_NPHS):
        cb = ph * _NITSH
        fire_g(0, 0, cb)
        fire_g(1, 1, cb)
        wait_g(0, 0, cb); fire_s(0, 0); fire_g(2, 2, cb)
        wait_g(1, 1, cb); fire_s(1, 1); wait_s(0, 0); fire_g(3, 0, cb)
        wait_g(2, 2, cb); fire_s(2, 2); wait_s(1, 1); fire_g(4, 1, cb)

        def trip(k, carry):
            q = 3 * k
            wait_g(q, 0, cb); fire_s(q, 0); wait_s(q - 1, 2)
            fire_g(q + 2, 2, cb)
            wait_g(q + 1, 1, cb); fire_s(q + 1, 1); wait_s(q, 0)
            fire_g(q + 3, 0, cb)
            wait_g(q + 2, 2, cb); fire_s(q + 2, 2); wait_s(q + 1, 1)
            fire_g(q + 4, 1, cb)
            return carry

        lax.fori_loop(1, (_NITSH - 5) // 3 + 1, trip, 0)
        wait_g(_NITSH - 2, 0, cb); fire_s(_NITSH - 2, 0); wait_s(_NITSH - 3, 2)
        wait_g(_NITSH - 1, 1, cb); fire_s(_NITSH - 1, 1)
        wait_s(_NITSH - 2, 0)
        wait_s(_NITSH - 1, 1)
        if ph + 1 < _NPHS:
            pltpu.sync_copy(row4_hbm.at[wid, ph + 1], ridx_half)
    plsc.subcore_barrier()
    pltpu.sync_copy(acc_sp.at[pl.ds(s * _RPW, _RPW)],
                    accp_hbm.at[c, pl.ds(s * _RPW, _RPW)])


@functools.lru_cache(maxsize=None)
def _make_spmm_kernel(interpret=False):
    return pl.kernel(
        _spmm_body,
        out_type=jax.ShapeDtypeStruct((_NC, _NP, _D), jnp.float32),
        mesh=_sc_mesh(),
        scratch_types=[
            pltpu.VMEM((_EPWP,), jnp.int32),
            pltpu.VMEM((_NITSH, _CHS), jnp.int32),
            pltpu.VMEM((_CHS, _D), jnp.float32),
            pltpu.VMEM((_CHS, _D), jnp.float32),
            pltpu.VMEM((_CHS, _D), jnp.float32),
            pltpu.SemaphoreType.DMA,
            pltpu.SemaphoreType.DMA,
            pltpu.SemaphoreType.DMA,
            pltpu.SemaphoreType.DMA,
            pltpu.SemaphoreType.DMA,
            pltpu.SemaphoreType.DMA,
            pltpu.SemaphoreType.DMA,
            pltpu.SemaphoreType.DMA,
            pltpu.VMEM_SHARED((_NP, _D), jnp.float32),
        ],
        interpret=interpret,
    )


_BLK = 1000


def _tc1_body(x_ref, w_ref, degp_ref, y_ref):
    i = pl.program_id(0)
    xw = jnp.dot(x_ref[...], w_ref[...], preferred_element_type=jnp.float32)
    deg = (degp_ref[0, pl.ds(i * _BLK, _BLK), 0:1]
           + degp_ref[1, pl.ds(i * _BLK, _BLK), 0:1] + 1.0)
    y_ref[...] = xw * (1.0 / jnp.sqrt(deg))


_tc1 = pl.pallas_call(
    _tc1_body,
    grid=(_N // _BLK,),
    in_specs=[
        pl.BlockSpec((_BLK, _D), lambda i: (i, 0)),
        pl.BlockSpec((_D, _D), lambda i: (0, 0)),
        pl.BlockSpec((_NC, _NP, _DW), lambda i: (0, 0, 0)),
    ],
    out_specs=pl.BlockSpec((_BLK, _D), lambda i: (i, 0)),
    out_shape=jax.ShapeDtypeStruct((_N, _D), jnp.float32),
)


def _tc2_body(accp_ref, y_ref, degp_ref, bias_ref, out_ref):
    i = pl.program_id(0)
    sl = pl.ds(i * _BLK, _BLK)
    acc = accp_ref[0, sl, :] + accp_ref[1, sl, :] + y_ref[...]
    deg = degp_ref[0, sl, 0:1] + degp_ref[1, sl, 0:1] + 1.0
    out_ref[...] = acc * (1.0 / jnp.sqrt(deg)) + bias_ref[...]


_tc2 = pl.pallas_call(
    _tc2_body,
    grid=(_N // _BLK,),
    in_specs=[
        pl.BlockSpec((_NC, _NP, _D), lambda i: (0, 0, 0)),
        pl.BlockSpec((_BLK, _D), lambda i: (i, 0)),
        pl.BlockSpec((_NC, _NP, _DW), lambda i: (0, 0, 0)),
        pl.BlockSpec((1, _D), lambda i: (0, 0)),
    ],
    out_specs=pl.BlockSpec((_BLK, _D), lambda i: (i, 0)),
    out_shape=jax.ShapeDtypeStruct((_N, _D), jnp.float32),
)


def kernel(x, edge_index, W, bias):
    ei = edge_index.astype(jnp.int32)
    col4 = ei[1].reshape(_NW, _NPH, _NITH, _CH)
    rowp = jnp.concatenate(
        [ei[0].reshape(_NW, _EPW),
         jnp.full((_NW, _EPAD), _N, jnp.int32)], axis=1)
    colp = jnp.concatenate(
        [ei[1].reshape(_NW, _EPW),
         jnp.zeros((_NW, _EPAD), jnp.int32)], axis=1)
    row4s = rowp.reshape(_NW, _NPHS, _NITSH, _CHS)
    colflat = colp.reshape(_NW * _EPWP)
    ones_dw = jnp.ones((_CH, _DW), jnp.float32)
    zeros_d = jnp.zeros((_NP, _D), jnp.float32)
    degp = _make_deg_kernel()(col4, ones_dw, zeros_d)
    y = _tc1(x, W, degp)
    accp = _make_spmm_kernel()(row4s, colflat, y, zeros_d)
    return _tc2(accp, y, degp, bias.reshape(1, _D))


# speedup vs baseline: 1.9782x; 1.9782x over previous
import functools

import jax
import jax.numpy as jnp
from jax import lax
from jax.experimental import pallas as pl
from jax.experimental.pallas import tpu as pltpu
from jax.experimental.pallas import tpu_sc as plsc

_N = 10000
_E = 320000
_D = 128
_NC = 2
_NS = 16
_NW = _NC * _NS
_EPW = _E // _NW
_CH = 40
_NIT = _EPW // _CH
_NPH = 2
_NITH = _NIT // _NPH
_NP = 10240
_RPW = _NP // _NS
_DW = 128
_DO = 16

@functools.lru_cache(maxsize=None)
def _sc_mesh():
    return plsc.VectorSubcoreMesh(core_axis_name="c", subcore_axis_name="s",
                                  num_cores=_NC, num_subcores=_NS)


def _deg_body(col4_hbm, ones_hbm, zeros_hbm, degp_hbm,
              cidx_half, ones_v, isem, ssem, deg_sp):
    c = lax.axis_index("c")
    s = lax.axis_index("s")
    wid = s * _NC + c
    pltpu.sync_copy(zeros_hbm.at[pl.ds(s * _RPW, _RPW)],
                    deg_sp.at[pl.ds(s * _RPW, _RPW)])
    pltpu.sync_copy(ones_hbm, ones_v)
    pltpu.sync_copy(col4_hbm.at[wid, 0], cidx_half)
    plsc.subcore_barrier()
    for ph in range(_NPH):
        pltpu.async_copy(ones_v, deg_sp.at[cidx_half.at[0]], ssem, add=True)
        pltpu.async_copy(ones_v, deg_sp.at[cidx_half.at[1]], ssem, add=True)

        def step(i, carry):
            pltpu.make_async_copy(ones_v, deg_sp.at[cidx_half.at[i]], ssem).wait()
            pltpu.async_copy(ones_v, deg_sp.at[cidx_half.at[i]], ssem, add=True)
            return carry

        lax.fori_loop(2, _NITH, step, 0)
        pltpu.make_async_copy(ones_v, deg_sp.at[cidx_half.at[0]], ssem).wait()
        pltpu.make_async_copy(ones_v, deg_sp.at[cidx_half.at[0]], ssem).wait()
        if ph + 1 < _NPH:
            pltpu.sync_copy(col4_hbm.at[wid, ph + 1], cidx_half)
    plsc.subcore_barrier()
    pltpu.sync_copy(deg_sp.at[pl.ds(s * _RPW, _RPW)],
                    degp_hbm.at[c, pl.ds(s * _RPW, _RPW)])


@functools.lru_cache(maxsize=None)
def _make_deg_kernel(interpret=False):
    return pl.kernel(
        _deg_body,
        out_type=jax.ShapeDtypeStruct((_NC, _NP, _DW), jnp.float32),
        mesh=_sc_mesh(),
        scratch_types=[
            pltpu.VMEM((_NITH, _CH), jnp.int32),
            pltpu.VMEM((_CH, _DW), jnp.float32),
            pltpu.SemaphoreType.DMA,
            pltpu.SemaphoreType.DMA,
            pltpu.VMEM_SHARED((_NP, _DW), jnp.float32),
        ],
        interpret=interpret,
    )


def _spmm_body(row4_hbm, col_hbm, y_hbm, zeros_hbm, accp_hbm,
               cidx_all, ridx_half, rows0, rows1, rows2,
               isem0, isem1, gs0, gs1, gs2, ss0, ss1, ss2, acc_sp):
    c = lax.axis_index("c")
    s = lax.axis_index("s")
    wid = s * _NC + c
    pltpu.async_copy(col_hbm.at[pl.ds(wid * _EPW, _EPW)], cidx_all, isem0)
    pltpu.async_copy(row4_hbm.at[wid, 0], ridx_half, isem1)
    pltpu.sync_copy(zeros_hbm.at[pl.ds(s * _RPW, _RPW)],
                    acc_sp.at[pl.ds(s * _RPW, _RPW)])
    pltpu.make_async_copy(col_hbm.at[pl.ds(wid * _EPW, _EPW)], cidx_all,
                          isem0).wait()
    pltpu.make_async_copy(row4_hbm.at[wid, 0], ridx_half, isem1).wait()
    plsc.subcore_barrier()

    rows = (rows0, rows1, rows2)
    gsem = (gs0, gs1, gs2)
    ssem = (ss0, ss1, ss2)

    def gidx(k):
        return cidx_all.at[pl.ds(k * _CH, _CH)]

    def fire_g(q, b, cb):
        pltpu.async_copy(y_hbm.at[gidx(cb + q)], rows[b], gsem[b])

    def wait_g(q, b, cb):
        pltpu.make_async_copy(y_hbm.at[gidx(cb + q)], rows[b], gsem[b]).wait()

    def fire_s(i, b):
        pltpu.async_copy(rows[b], acc_sp.at[ridx_half.at[i]], ssem[b],
                         add=True)

    def wait_s(i, b):
        pltpu.make_async_copy(rows[b], acc_sp.at[ridx_half.at[i]],
                              ssem[b]).wait()

    for ph in range(_NPH):
        cb = ph * _NITH
        fire_g(0, 0, cb)
        fire_g(1, 1, cb)
        wait_g(0, 0, cb); fire_s(0, 0); fire_g(2, 2, cb)
        wait_g(1, 1, cb); fire_s(1, 1); wait_s(0, 0); fire_g(3, 0, cb)
        wait_g(2, 2, cb); fire_s(2, 2); wait_s(1, 1); fire_g(4, 1, cb)

        def trip(k, carry):
            q = 3 * k
            wait_g(q, 0, cb); fire_s(q, 0); wait_s(q - 1, 2)
            fire_g(q + 2, 2, cb)
            wait_g(q + 1, 1, cb); fire_s(q + 1, 1); wait_s(q, 0)
            fire_g(q + 3, 0, cb)
            wait_g(q + 2, 2, cb); fire_s(q + 2, 2); wait_s(q + 1, 1)
            fire_g(q + 4, 1, cb)
            return carry

        lax.fori_loop(1, (_NITH - 5) // 3 + 1, trip, 0)
        wait_g(_NITH - 2, 0, cb); fire_s(_NITH - 2, 0); wait_s(_NITH - 3, 2)
        wait_g(_NITH - 1, 1, cb); fire_s(_NITH - 1, 1)
        wait_s(_NITH - 2, 0)
        wait_s(_NITH - 1, 1)
        if ph + 1 < _NPH:
            pltpu.sync_copy(row4_hbm.at[wid, ph + 1], ridx_half)
    plsc.subcore_barrier()
    pltpu.sync_copy(acc_sp.at[pl.ds(s * _RPW, _RPW)],
                    accp_hbm.at[c, pl.ds(s * _RPW, _RPW)])


@functools.lru_cache(maxsize=None)
def _make_spmm_kernel(interpret=False):
    return pl.kernel(
        _spmm_body,
        out_type=jax.ShapeDtypeStruct((_NC, _NP, _D), jnp.float32),
        mesh=_sc_mesh(),
        scratch_types=[
            pltpu.VMEM((_EPW,), jnp.int32),
            pltpu.VMEM((_NITH, _CH), jnp.int32),
            pltpu.VMEM((_CH, _D), jnp.float32),
            pltpu.VMEM((_CH, _D), jnp.float32),
            pltpu.VMEM((_CH, _D), jnp.float32),
            pltpu.SemaphoreType.DMA,
            pltpu.SemaphoreType.DMA,
            pltpu.SemaphoreType.DMA,
            pltpu.SemaphoreType.DMA,
            pltpu.SemaphoreType.DMA,
            pltpu.SemaphoreType.DMA,
            pltpu.SemaphoreType.DMA,
            pltpu.SemaphoreType.DMA,
            pltpu.VMEM_SHARED((_NP, _D), jnp.float32),
        ],
        interpret=interpret,
    )


_BLK = 1000


def _tc1_body(x_ref, w_ref, degp_ref, y_ref):
    i = pl.program_id(0)
    xw = jnp.dot(x_ref[...], w_ref[...], preferred_element_type=jnp.float32)
    deg = (degp_ref[0, pl.ds(i * _BLK, _BLK), 0:1]
           + degp_ref[1, pl.ds(i * _BLK, _BLK), 0:1] + 1.0)
    y_ref[...] = xw * (1.0 / jnp.sqrt(deg))


_tc1 = pl.pallas_call(
    _tc1_body,
    grid=(_N // _BLK,),
    in_specs=[
        pl.BlockSpec((_BLK, _D), lambda i: (i, 0)),
        pl.BlockSpec((_D, _D), lambda i: (0, 0)),
        pl.BlockSpec((_NC, _NP, _DW), lambda i: (0, 0, 0)),
    ],
    out_specs=pl.BlockSpec((_BLK, _D), lambda i: (i, 0)),
    out_shape=jax.ShapeDtypeStruct((_N, _D), jnp.float32),
)


def _tc2_body(accp_ref, y_ref, degp_ref, bias_ref, out_ref):
    i = pl.program_id(0)
    sl = pl.ds(i * _BLK, _BLK)
    acc = accp_ref[0, sl, :] + accp_ref[1, sl, :] + y_ref[...]
    deg = degp_ref[0, sl, 0:1] + degp_ref[1, sl, 0:1] + 1.0
    out_ref[...] = acc * (1.0 / jnp.sqrt(deg)) + bias_ref[...]


_tc2 = pl.pallas_call(
    _tc2_body,
    grid=(_N // _BLK,),
    in_specs=[
        pl.BlockSpec((_NC, _NP, _D), lambda i: (0, 0, 0)),
        pl.BlockSpec((_BLK, _D), lambda i: (i, 0)),
        pl.BlockSpec((_NC, _NP, _DW), lambda i: (0, 0, 0)),
        pl.BlockSpec((1, _D), lambda i: (0, 0)),
    ],
    out_specs=pl.BlockSpec((_BLK, _D), lambda i: (i, 0)),
    out_shape=jax.ShapeDtypeStruct((_N, _D), jnp.float32),
)


def kernel(x, edge_index, W, bias):
    ei = edge_index.astype(jnp.int32)
    row4 = ei[0].reshape(_NW, _NPH, _NITH, _CH)
    col = ei[1]
    col4 = col.reshape(_NW, _NPH, _NITH, _CH)
    ones_dw = jnp.ones((_CH, _DW), jnp.float32)
    zeros_d = jnp.zeros((_NP, _D), jnp.float32)
    degp = _make_deg_kernel()(col4, ones_dw, zeros_d)
    y = _tc1(x, W, degp)
    accp = _make_spmm_kernel()(row4, col, y, zeros_d)
    return _tc2(accp, y, degp, bias.reshape(1, _D))


# speedup vs baseline: 1.9807x; 1.0013x over previous
import functools

import jax
import jax.numpy as jnp
from jax import lax
from jax.experimental import pallas as pl
from jax.experimental.pallas import tpu as pltpu
from jax.experimental.pallas import tpu_sc as plsc

_N = 10000
_E = 320000
_D = 128
_NC = 2
_NS = 16
_NW = _NC * _NS
_EPW = _E // _NW
_CH = 40
_NIT = _EPW // _CH
_NPH = 2
_NITH = _NIT // _NPH
_NP = 10240
_RPW = _NP // _NS
_DW = 128

@functools.lru_cache(maxsize=None)
def _sc_mesh():
    return plsc.VectorSubcoreMesh(core_axis_name="c", subcore_axis_name="s",
                                  num_cores=_NC, num_subcores=_NS)


def _deg_body(col4_hbm, ones_hbm, zeros_hbm, degp_hbm,
              cidx_half, ones_v, isem, ssem, deg_sp):
    c = lax.axis_index("c")
    s = lax.axis_index("s")
    wid = s * _NC + c
    pltpu.sync_copy(zeros_hbm.at[pl.ds(s * _RPW, _RPW)],
                    deg_sp.at[pl.ds(s * _RPW, _RPW)])
    pltpu.sync_copy(ones_hbm, ones_v)
    pltpu.sync_copy(col4_hbm.at[wid, 0], cidx_half)
    plsc.subcore_barrier()
    for ph in range(_NPH):
        pltpu.async_copy(ones_v, deg_sp.at[cidx_half.at[0]], ssem, add=True)
        pltpu.async_copy(ones_v, deg_sp.at[cidx_half.at[1]], ssem, add=True)

        def step(i, carry):
            pltpu.make_async_copy(ones_v, deg_sp.at[cidx_half.at[i]], ssem).wait()
            pltpu.async_copy(ones_v, deg_sp.at[cidx_half.at[i]], ssem, add=True)
            return carry

        lax.fori_loop(2, _NITH, step, 0)
        pltpu.make_async_copy(ones_v, deg_sp.at[cidx_half.at[0]], ssem).wait()
        pltpu.make_async_copy(ones_v, deg_sp.at[cidx_half.at[0]], ssem).wait()
        if ph + 1 < _NPH:
            pltpu.sync_copy(col4_hbm.at[wid, ph + 1], cidx_half)
    plsc.subcore_barrier()
    pltpu.sync_copy(deg_sp.at[pl.ds(s * _RPW, _RPW)],
                    degp_hbm.at[c, pl.ds(s * _RPW, _RPW)])


@functools.lru_cache(maxsize=None)
def _make_deg_kernel(interpret=False):
    return pl.kernel(
        _deg_body,
        out_type=jax.ShapeDtypeStruct((_NC, _NP, _DW), jnp.float32),
        mesh=_sc_mesh(),
        scratch_types=[
            pltpu.VMEM((_NITH, _CH), jnp.int32),
            pltpu.VMEM((_CH, _DW), jnp.float32),
            pltpu.SemaphoreType.DMA,
            pltpu.SemaphoreType.DMA,
            pltpu.VMEM_SHARED((_NP, _DW), jnp.float32),
        ],
        interpret=interpret,
    )


def _spmm_body(row4_hbm, col_hbm, y_hbm, zeros_hbm, accp_hbm,
               cidx_all, ridx_half, rows0, rows1, rows2,
               isem0, isem1, gs0, gs1, gs2, ss0, ss1, ss2, acc_sp):
    c = lax.axis_index("c")
    s = lax.axis_index("s")
    wid = s * _NC + c
    pltpu.async_copy(col_hbm.at[pl.ds(wid * _EPW, _EPW)], cidx_all, isem0)
    pltpu.async_copy(row4_hbm.at[wid, 0], ridx_half, isem1)
    pltpu.sync_copy(zeros_hbm.at[pl.ds(s * _RPW, _RPW)],
                    acc_sp.at[pl.ds(s * _RPW, _RPW)])
    pltpu.make_async_copy(col_hbm.at[pl.ds(wid * _EPW, _EPW)], cidx_all,
                          isem0).wait()
    pltpu.make_async_copy(row4_hbm.at[wid, 0], ridx_half, isem1).wait()
    plsc.subcore_barrier()

    rows = (rows0, rows1, rows2)
    gsem = (gs0, gs1, gs2)
    ssem = (ss0, ss1, ss2)

    def gidx(k):
        return cidx_all.at[pl.ds(k * _CH, _CH)]

    def fire_g(q, b, cb):
        pltpu.async_copy(y_hbm.at[gidx(cb + q)], rows[b], gsem[b])

    def wait_g(q, b, cb):
        pltpu.make_async_copy(y_hbm.at[gidx(cb + q)], rows[b], gsem[b]).wait()

    def fire_s(i, b):
        pltpu.async_copy(rows[b], acc_sp.at[ridx_half.at[i]], ssem[b],
                         add=True)

    def wait_s(i, b):
        pltpu.make_async_copy(rows[b], acc_sp.at[ridx_half.at[i]],
                              ssem[b]).wait()

    for ph in range(_NPH):
        cb = ph * _NITH
        fire_g(0, 0, cb)
        fire_g(1, 1, cb)
        wait_g(0, 0, cb); fire_s(0, 0); fire_g(2, 2, cb)
        wait_g(1, 1, cb); fire_s(1, 1); wait_s(0, 0); fire_g(3, 0, cb)
        wait_g(2, 2, cb); fire_s(2, 2); wait_s(1, 1); fire_g(4, 1, cb)

        def trip(k, carry):
            q = 3 * k
            wait_g(q, 0, cb); fire_s(q, 0); wait_s(q - 1, 2)
            fire_g(q + 2, 2, cb)
            wait_g(q + 1, 1, cb); fire_s(q + 1, 1); wait_s(q, 0)
            fire_g(q + 3, 0, cb)
            wait_g(q + 2, 2, cb); fire_s(q + 2, 2); wait_s(q + 1, 1)
            fire_g(q + 4, 1, cb)
            return carry

        lax.fori_loop(1, (_NITH - 5) // 3 + 1, trip, 0)
        wait_g(_NITH - 2, 0, cb); fire_s(_NITH - 2, 0); wait_s(_NITH - 3, 2)
        wait_g(_NITH - 1, 1, cb); fire_s(_NITH - 1, 1)
        wait_s(_NITH - 2, 0)
        wait_s(_NITH - 1, 1)
        if ph + 1 < _NPH:
            pltpu.sync_copy(row4_hbm.at[wid, ph + 1], ridx_half)
    plsc.subcore_barrier()
    pltpu.sync_copy(acc_sp.at[pl.ds(s * _RPW, _RPW)],
                    accp_hbm.at[c, pl.ds(s * _RPW, _RPW)])


@functools.lru_cache(maxsize=None)
def _make_spmm_kernel(interpret=False):
    return pl.kernel(
        _spmm_body,
        out_type=jax.ShapeDtypeStruct((_NC, _NP, _D), jnp.float32),
        mesh=_sc_mesh(),
        scratch_types=[
            pltpu.VMEM((_EPW,), jnp.int32),
            pltpu.VMEM((_NITH, _CH), jnp.int32),
            pltpu.VMEM((_CH, _D), jnp.float32),
            pltpu.VMEM((_CH, _D), jnp.float32),
            pltpu.VMEM((_CH, _D), jnp.float32),
            pltpu.SemaphoreType.DMA,
            pltpu.SemaphoreType.DMA,
            pltpu.SemaphoreType.DMA,
            pltpu.SemaphoreType.DMA,
            pltpu.SemaphoreType.DMA,
            pltpu.SemaphoreType.DMA,
            pltpu.SemaphoreType.DMA,
            pltpu.SemaphoreType.DMA,
            pltpu.VMEM_SHARED((_NP, _D), jnp.float32),
        ],
        interpret=interpret,
    )


_BLK = 1000


def _tc1_body(x_ref, w_ref, degp_ref, y_ref):
    i = pl.program_id(0)
    xw = jnp.dot(x_ref[...], w_ref[...], preferred_element_type=jnp.float32)
    deg = (degp_ref[0, pl.ds(i * _BLK, _BLK), 0:1]
           + degp_ref[1, pl.ds(i * _BLK, _BLK), 0:1] + 1.0)
    y_ref[...] = xw * (1.0 / jnp.sqrt(deg))


_tc1 = pl.pallas_call(
    _tc1_body,
    grid=(_N // _BLK,),
    in_specs=[
        pl.BlockSpec((_BLK, _D), lambda i: (i, 0)),
        pl.BlockSpec((_D, _D), lambda i: (0, 0)),
        pl.BlockSpec((_NC, _NP, _DW), lambda i: (0, 0, 0)),
    ],
    out_specs=pl.BlockSpec((_BLK, _D), lambda i: (i, 0)),
    out_shape=jax.ShapeDtypeStruct((_N, _D), jnp.float32),
)


def _tc2_body(accp_ref, y_ref, degp_ref, bias_ref, out_ref):
    i = pl.program_id(0)
    sl = pl.ds(i * _BLK, _BLK)
    acc = accp_ref[0, sl, :] + accp_ref[1, sl, :] + y_ref[...]
    deg = degp_ref[0, sl, 0:1] + degp_ref[1, sl, 0:1] + 1.0
    out_ref[...] = acc * (1.0 / jnp.sqrt(deg)) + bias_ref[...]


_tc2 = pl.pallas_call(
    _tc2_body,
    grid=(_N // _BLK,),
    in_specs=[
        pl.BlockSpec((_NC, _NP, _D), lambda i: (0, 0, 0)),
        pl.BlockSpec((_BLK, _D), lambda i: (i, 0)),
        pl.BlockSpec((_NC, _NP, _DW), lambda i: (0, 0, 0)),
        pl.BlockSpec((1, _D), lambda i: (0, 0)),
    ],
    out_specs=pl.BlockSpec((_BLK, _D), lambda i: (i, 0)),
    out_shape=jax.ShapeDtypeStruct((_N, _D), jnp.float32),
)


def kernel(x, edge_index, W, bias):
    ei = edge_index.astype(jnp.int32)
    row4 = ei[0].reshape(_NW, _NPH, _NITH, _CH)
    col = ei[1]
    col4 = col.reshape(_NW, _NPH, _NITH, _CH)
    ones_dw = jnp.ones((_CH, _DW), jnp.float32)
    zeros_d = jnp.zeros((_NP, _D), jnp.float32)
    degp = _make_deg_kernel()(col4, ones_dw, zeros_d)
    y = _tc1(x, W, degp)
    accp = _make_spmm_kernel()(row4, col, y, zeros_d)
    return _tc2(accp, y, degp, bias.reshape(1, _D))
